# Initial kernel scaffold; baseline (speedup 1.0000x reference)
import jax
import jax.numpy as jnp
from jax.experimental import pallas as pl


def kernel(feat, edge_index, W1, attn_l1, attn_r1, b1, W2, attn_l2, attn_r2, b2):
    raise NotImplementedError("write your pallas kernel here")


# speedup vs baseline: 40.5196x; 40.5196x over previous
import jax
import jax.numpy as jnp
from jax import lax
from jax.experimental import pallas as pl
from jax.experimental.pallas import tpu as pltpu
from jax.experimental.pallas import tpu_sc as plsc

_N = 10000
_E = 320000
_D = 128
_HD = _D // 2
_NEG = 0.2
_NT = 16
_EPT = _E // _NT
_CH = 80
_NCH = _EPT // _CH
_AW = 80
_SL = 50
_NSL = _NCH // _SL
_RPT = _N // _NT
_ZR = 125
_BN = 1000
_GB = _N // _BN


def _eler_block(h, al, ar):
    el = jnp.sum(h * al, axis=1)
    er = jnp.sum(h * ar, axis=1)
    row = lax.broadcasted_iota(jnp.int32, (8, _BN), 0)
    return jnp.where(row == 0, el[None, :],
                     jnp.where(row == 1, er[None, :], 0.0))[None]


def _tc_mm_body(x_ref, wl_ref, wh_ref, al_ref, ar_ref, h_ref, eler_ref):
    x = x_ref[...]
    hl = jnp.dot(x, wl_ref[...], preferred_element_type=jnp.float32)
    hh = jnp.dot(x, wh_ref[...], preferred_element_type=jnp.float32)
    h_ref[0, :, :] = hl
    h_ref[1, :, :] = hh
    h = jnp.concatenate([hl, hh], axis=1)
    eler_ref[...] = _eler_block(h, al_ref[...], ar_ref[...])


def _tc_mm(x, w, al, ar):
    return pl.pallas_call(
        _tc_mm_body,
        grid=(_GB,),
        in_specs=[
            pl.BlockSpec((_BN, _D), lambda i: (i, 0)),
            pl.BlockSpec((_D, _HD), lambda i: (0, 0)),
            pl.BlockSpec((_D, _HD), lambda i: (0, 0)),
            pl.BlockSpec((1, _D), lambda i: (0, 0)),
            pl.BlockSpec((1, _D), lambda i: (0, 0)),
        ],
        out_specs=[
            pl.BlockSpec((2, _BN, _HD), lambda i: (0, i, 0)),
            pl.BlockSpec((1, 8, _BN), lambda i: (i, 0, 0)),
        ],
        out_shape=[
            jax.ShapeDtypeStruct((2, _N, _HD), jnp.float32),
            jax.ShapeDtypeStruct((_GB, 8, _BN), jnp.float32),
        ],
    )(x, w[:, :_HD], w[:, _HD:], al, ar)


def _tc_comb_mm_body(u0_ref, u1_ref, s_ref, b_ref, wl_ref, wh_ref, al_ref,
                     ar_ref, h1_ref, h_ref, eler_ref):
    s = s_ref[...] + 1e-9
    u = jnp.concatenate([u0_ref[...], u1_ref[...]], axis=1)
    h1 = jnp.maximum(u / s + b_ref[...], 0.0)
    h1_ref[...] = h1
    hl = jnp.dot(h1, wl_ref[...], preferred_element_type=jnp.float32)
    hh = jnp.dot(h1, wh_ref[...], preferred_element_type=jnp.float32)
    h_ref[0, :, :] = hl
    h_ref[1, :, :] = hh
    h = jnp.concatenate([hl, hh], axis=1)
    eler_ref[...] = _eler_block(h, al_ref[...], ar_ref[...])


def _tc_comb_mm(u0, u1, s, b, w, al, ar):
    return pl.pallas_call(
        _tc_comb_mm_body,
        grid=(_GB,),
        in_specs=[
            pl.BlockSpec((_BN, _HD), lambda i: (i, 0)),
            pl.BlockSpec((_BN, _HD), lambda i: (i, 0)),
            pl.BlockSpec((_BN, 1), lambda i: (i, 0)),
            pl.BlockSpec((1, _D), lambda i: (0, 0)),
            pl.BlockSpec((_D, _HD), lambda i: (0, 0)),
            pl.BlockSpec((_D, _HD), lambda i: (0, 0)),
            pl.BlockSpec((1, _D), lambda i: (0, 0)),
            pl.BlockSpec((1, _D), lambda i: (0, 0)),
        ],
        out_specs=[
            pl.BlockSpec((_BN, _D), lambda i: (i, 0)),
            pl.BlockSpec((2, _BN, _HD), lambda i: (0, i, 0)),
            pl.BlockSpec((1, 8, _BN), lambda i: (i, 0, 0)),
        ],
        out_shape=[
            jax.ShapeDtypeStruct((_N, _D), jnp.float32),
            jax.ShapeDtypeStruct((2, _N, _HD), jnp.float32),
            jax.ShapeDtypeStruct((_GB, 8, _BN), jnp.float32),
        ],
    )(u0, u1, s, b, w[:, :_HD], w[:, _HD:], al, ar)


def _tc_comb_body(u0_ref, u1_ref, s_ref, b_ref, h2_ref):
    s = s_ref[...] + 1e-9
    u = jnp.concatenate([u0_ref[...], u1_ref[...]], axis=1)
    h2_ref[...] = jnp.maximum(u / s + b_ref[...], 0.0)


def _tc_comb(u0, u1, s, b):
    return pl.pallas_call(
        _tc_comb_body,
        grid=(_GB,),
        in_specs=[
            pl.BlockSpec((_BN, _HD), lambda i: (i, 0)),
            pl.BlockSpec((_BN, _HD), lambda i: (i, 0)),
            pl.BlockSpec((_BN, 1), lambda i: (i, 0)),
            pl.BlockSpec((1, _D), lambda i: (0, 0)),
        ],
        out_specs=pl.BlockSpec((_BN, _D), lambda i: (i, 0)),
        out_shape=jax.ShapeDtypeStruct((_N, _D), jnp.float32),
    )(u0, u1, s, b)


def _sc_msg_body(h_hbm, eler_hbm, src_hbm, dst_hbm, out_hbm,
                 src_v, dst_v, p_v, el_v, er_v, gbuf, ebuf, zbuf, acc,
                 gsem0, gsem1, ssem):
    c = lax.axis_index("c")
    s = lax.axis_index("s")

    zero16 = jnp.zeros((16,), jnp.float32)

    def _zrow(r, carry):
        for j in range(_AW // 16):
            zbuf[r, pl.ds(j * 16, 16)] = zero16
        return carry
    lax.fori_loop(0, _ZR, _zrow, 0)
    for k in range(_RPT // _ZR):
        pltpu.sync_copy(zbuf, acc.at[pl.ds(s * _RPT + k * _ZR, _ZR)])
    plsc.subcore_barrier()

    for g in range(_GB):
        pltpu.sync_copy(eler_hbm.at[g, 0], el_v.at[pl.ds(g * _BN, _BN)])
        pltpu.sync_copy(eler_hbm.at[g, 1], er_v.at[pl.ds(g * _BN, _BN)])

    lane0 = jnp.where(lax.iota(jnp.int32, 16) == 0, 1.0, 0.0)
    zeros16i = jnp.zeros((16,), jnp.int32)
    gsems = (gsem0, gsem1)

    def _slab(sl, carry):
        pltpu.sync_copy(src_hbm.at[s, pl.ds(sl * _SL, _SL)], src_v)
        pltpu.sync_copy(dst_hbm.at[s, pl.ds(sl * _SL, _SL)], dst_v)
        pltpu.async_copy(h_hbm.at[c].at[src_v.at[0]], gbuf.at[0], gsem0)

        def _pair(cj, carry1):
            for b in range(2):
                ci = 2 * cj + b
                nb = 1 - b

                @pl.when(ci + 1 < _SL)
                def _prefetch():
                    pltpu.async_copy(h_hbm.at[c].at[src_v.at[ci + 1]],
                                     gbuf.at[nb], gsems[nb])

                for k in range(_CH // 16):
                    sv = src_v[ci, pl.ds(k * 16, 16)]
                    dv = dst_v[ci, pl.ds(k * 16, 16)]
                    e = (plsc.load_gather(el_v, [sv])
                         + plsc.load_gather(er_v, [dv]))
                    e = jnp.where(e > 0.0, e, _NEG * e)
                    p_v[ci, pl.ds(k * 16, 16)] = jnp.exp(e)

                pltpu.make_async_copy(h_hbm.at[c].at[src_v.at[ci]],
                                      gbuf.at[b], gsems[b]).wait()

                @pl.when(sl * _SL + ci >= 2)
                def _drain():
                    pltpu.make_async_copy(ebuf.at[b],
                                          acc.at[pl.ds(0, _CH)], ssem).wait()

                @plsc.parallel_loop(0, _CH, unroll=4)
                def _edge(ei):
                    pi = plsc.load_gather(p_v, [zeros16i + ci, zeros16i + ei])
                    for j in range(_HD // 16):
                        ebuf[b, ei, pl.ds(j * 16, 16)] = (
                            gbuf[b, ei, pl.ds(j * 16, 16)] * pi)
                    ebuf[b, ei, pl.ds(_HD, 16)] = pi * lane0

                pltpu.async_copy(ebuf.at[b], acc.at[dst_v.at[ci]], ssem,
                                 add=True)
            return carry1
        lax.fori_loop(0, _SL // 2, _pair, 0)
        return carry
    lax.fori_loop(0, _NSL, _slab, 0)

    for _ in range(2):
        pltpu.make_async_copy(ebuf.at[0], acc.at[pl.ds(0, _CH)], ssem).wait()

    plsc.subcore_barrier()
    pltpu.sync_copy(acc.at[pl.ds(s * _RPT, _RPT)],
                    out_hbm.at[c, pl.ds(s * _RPT, _RPT)])


_sc_msg = pl.kernel(
    _sc_msg_body,
    out_type=jax.ShapeDtypeStruct((2, _N, _AW), jnp.float32),
    mesh=plsc.VectorSubcoreMesh(core_axis_name="c", subcore_axis_name="s",
                                num_cores=2, num_subcores=16),
    compiler_params=pltpu.CompilerParams(use_tc_tiling_on_sc=False,
                                         needs_layout_passes=False),
    scratch_types=[
        pltpu.VMEM((_SL, _CH), jnp.int32),
        pltpu.VMEM((_SL, _CH), jnp.int32),
        pltpu.VMEM((_SL, _CH), jnp.float32),
        pltpu.VMEM((_N,), jnp.float32),
        pltpu.VMEM((_N,), jnp.float32),
        pltpu.VMEM((2, _CH, _HD), jnp.float32),
        pltpu.VMEM((2, _CH, _AW), jnp.float32),
        pltpu.VMEM((_ZR, _AW), jnp.float32),
        pltpu.VMEM_SHARED((_N, _AW), jnp.float32),
        pltpu.SemaphoreType.DMA,
        pltpu.SemaphoreType.DMA,
        pltpu.SemaphoreType.DMA,
    ],
)


def kernel(feat, edge_index, W1, attn_l1, attn_r1, b1, W2, attn_l2, attn_r2, b2):
    src = edge_index[0].reshape(_NT, _NCH, _CH)
    dst = edge_index[1].reshape(_NT, _NCH, _CH)

    h, eler = _tc_mm(feat, W1, attn_l1, attn_r1)
    U = _sc_msg(h, eler, src, dst)
    h1, hmid, eler2 = _tc_comb_mm(
        U[0, :, :_HD], U[1, :, :_HD], U[0, :, _HD:_HD + 1],
        b1.reshape(1, _D), W2, attn_l2, attn_r2)
    U2 = _sc_msg(hmid, eler2, src, dst)
    h2 = _tc_comb(
        U2[0, :, :_HD], U2[1, :, :_HD], U2[0, :, _HD:_HD + 1],
        b2.reshape(1, _D))
    return (h1, h2)
